# baseline (speedup 1.0000x reference)
import functools

import jax
import jax.numpy as jnp
from jax import lax
from jax.experimental import pallas as pl
from jax.experimental.pallas import tpu as pltpu
from jax.experimental.pallas import tpu_sc as plsc

N = 10000
E = 320000
D = 128
H = 128

NC = 2
NS = 16
NW = NC * NS
CHUNK = 128

K = 2
EK = E // K
NCHUNK = EK // CHUNK

GCPT = (((NCHUNK + NW - 1) // NW) + 1) // 2 * 2
GID = GCPT * CHUNK
SBASE = NCHUNK // NW
SREM = NCHUNK % NW
SGROUPS = (SBASE + 2) // 2

BN = 2000
BE = 2000

_MESH = plsc.VectorSubcoreMesh(
    core_axis_name="c", subcore_axis_name="s", num_cores=NC, num_subcores=NS)


def _pre_body(node_ref, w_ref, ps_ref, pr_ref):
    x = node_ref[...]
    ps_ref[...] = jnp.dot(x, w_ref[0:D, :], preferred_element_type=jnp.float32)
    pr_ref[...] = jnp.dot(x, w_ref[D:2 * D, :], preferred_element_type=jnp.float32)


def _pre_call(node, w_sr):
    return pl.pallas_call(
        _pre_body,
        grid=(N // BN,),
        in_specs=[
            pl.BlockSpec((BN, D), lambda i: (i, 0)),
            pl.BlockSpec((2 * D, H), lambda i: (0, 0)),
        ],
        out_specs=[
            pl.BlockSpec((BN, H), lambda i: (i, 0)),
            pl.BlockSpec((BN, H), lambda i: (i, 0)),
        ],
        out_shape=[
            jax.ShapeDtypeStruct((N, H), jnp.float32),
            jax.ShapeDtypeStruct((N, H), jnp.float32),
        ],
    )(node, w_sr)


HW = H // 2


@functools.partial(
    pl.kernel,
    out_type=jax.ShapeDtypeStruct((EK, HW), jnp.int32),
    mesh=_MESH,
    scratch_types=[
        pltpu.VMEM((GID,), jnp.int32),
        pltpu.VMEM((GID,), jnp.int32),
        pltpu.VMEM((CHUNK, H), jnp.float32),
        pltpu.VMEM((CHUNK, H), jnp.float32),
        pltpu.VMEM((CHUNK, HW), jnp.int32),
        pltpu.VMEM((CHUNK, H), jnp.float32),
        pltpu.VMEM((CHUNK, H), jnp.float32),
        pltpu.VMEM((CHUNK, HW), jnp.int32),
        pltpu.SemaphoreType.DMA,
        pltpu.SemaphoreType.DMA,
        pltpu.SemaphoreType.DMA,
        pltpu.SemaphoreType.DMA,
    ],
)
def _sc_gather(ps_hbm, pr_hbm, s_hbm, r_hbm, g_hbm,
               idxs, idxr, bufa0, bufb0, out0, bufa1, bufb1, out1,
               gsem0, gsem1, wsem0, wsem1):
    wid = lax.axis_index("s") * NC + lax.axis_index("c")
    start = (wid * (NCHUNK - GCPT)) // (NW - 1)
    pltpu.sync_copy(s_hbm.at[pl.ds(start * CHUNK, GID)], idxs)
    pltpu.sync_copy(r_hbm.at[pl.ds(start * CHUNK, GID)], idxr)

    bufa = (bufa0, bufa1)
    bufb = (bufb0, bufb1)
    outb = (out0, out1)
    gsem = (gsem0, gsem1)
    wsem = (wsem0, wsem1)

    def fire(i, b):
        pltpu.async_copy(ps_hbm.at[idxs.at[pl.ds(i * CHUNK, CHUNK)]],
                         bufa[b], gsem[b])
        pltpu.async_copy(pr_hbm.at[idxr.at[pl.ds(i * CHUNK, CHUNK)]],
                         bufb[b], gsem[b])

    def wait_gather(i, b):
        pltpu.make_async_copy(ps_hbm.at[idxs.at[pl.ds(i * CHUNK, CHUNK)]],
                              bufa[b], gsem[b]).wait()
        pltpu.make_async_copy(pr_hbm.at[idxr.at[pl.ds(i * CHUNK, CHUNK)]],
                              bufb[b], gsem[b]).wait()

    for b in range(2):
        fire(jnp.int32(b), b)

    def group(g, carry):
        for b in range(2):
            i = g * 2 + b
            wait_gather(i, b)

            @pl.when(g > 0)
            def _():
                pltpu.make_async_copy(
                    outb[b], g_hbm.at[pl.ds(0, CHUNK)], wsem[b]).wait()

            ob, ba, bb = outb[b], bufa[b], bufb[b]

            MHI = jnp.int32(-65536)
            RB = jnp.int32(32768)

            @plsc.parallel_loop(0, CHUNK, 1, unroll=2)
            def _(r):
                for c in range(HW // 16):
                    s0 = pl.ds(32 * c, 16)
                    s1 = pl.ds(32 * c + 16, 16)
                    a0 = lax.bitcast_convert_type(
                        ba[r, s0] + bb[r, s0], jnp.int32) + RB
                    a1 = lax.bitcast_convert_type(
                        ba[r, s1] + bb[r, s1], jnp.int32) + RB
                    ob[r, pl.ds(c * 16, 16)] = jnp.bitwise_or(
                        lax.shift_right_logical(a0, 16),
                        jnp.bitwise_and(a1, MHI))

            nxt = i + 2

            @pl.when(nxt < GCPT)
            def _():
                fire(nxt, b)

            pltpu.async_copy(
                ob, g_hbm.at[pl.ds((start + i) * CHUNK, CHUNK)], wsem[b])
        return carry

    lax.fori_loop(0, GCPT // 2, group, 0)
    for b in range(2):
        pltpu.make_async_copy(outb[b], g_hbm.at[pl.ds(0, CHUNK)],
                              wsem[b]).wait()


def _edge_body(gsum_ref, e_ref, w1x_ref, b1_ref, w2_ref, b2_ref,
               g_ref, b_ref, pre_ref, out_ref):
    e = e_ref[...]
    gi = gsum_ref[...]
    ev = lax.bitcast_convert_type(jnp.left_shift(gi, 16), jnp.float32)
    od = lax.bitcast_convert_type(
        jnp.bitwise_and(gi, jnp.int32(-65536)), jnp.float32)
    gs = jnp.concatenate([ev, od], axis=1)
    h1 = (gs
          + jnp.dot(e, w1x_ref[...], preferred_element_type=jnp.float32)
          + b1_ref[...])
    h1 = jnp.maximum(h1, 0.0)
    h2 = jnp.dot(h1, w2_ref[...], preferred_element_type=jnp.float32) + b2_ref[...]
    mu = jnp.mean(h2, axis=-1, keepdims=True)
    var = jnp.mean((h2 - mu) ** 2, axis=-1, keepdims=True)
    y = (h2 - mu) / jnp.sqrt(var + 1e-5) * g_ref[...] + b_ref[...]
    pre_ref[...] = y
    out_ref[...] = y + e


def _edge_body_alias(gsum_ref, e_ref, w1x_ref, b1_ref, w2_ref, b2_ref,
                     g_ref, b_ref, prev_ref, pre_ref, out_ref):
    del prev_ref
    _edge_body(gsum_ref, e_ref, w1x_ref, b1_ref, w2_ref, b2_ref,
               g_ref, b_ref, pre_ref, out_ref)


def _edge_call(k, gsum, edge_k, w1x, b1, w2, b2, g, b, prev):
    full = lambda i: (0, 0)
    blk = lambda i: (i, 0)
    off = k * (EK // BE)
    out_blk = lambda i, _off=off: (_off + i, 0)
    in_specs = [
        pl.BlockSpec((BE, HW), blk),
        pl.BlockSpec((BE, D), blk),
        pl.BlockSpec((D, H), full),
        pl.BlockSpec((1, H), full),
        pl.BlockSpec((H, D), full),
        pl.BlockSpec((1, D), full),
        pl.BlockSpec((1, D), full),
        pl.BlockSpec((1, D), full),
    ]
    args = [gsum, edge_k, w1x, b1, w2, b2, g, b]
    body = _edge_body
    io_aliases = {}
    if prev is not None:
        in_specs.append(pl.BlockSpec((8, D), full))
        args.append(prev)
        body = _edge_body_alias
        io_aliases = {8: 1}
    return pl.pallas_call(
        body,
        grid=(EK // BE,),
        in_specs=in_specs,
        out_specs=[
            pl.BlockSpec((BE, D), blk),
            pl.BlockSpec((BE, D), out_blk),
        ],
        out_shape=[
            jax.ShapeDtypeStruct((EK, D), jnp.float32),
            jax.ShapeDtypeStruct((E, D), jnp.float32),
        ],
        input_output_aliases=io_aliases,
    )(*args)


@functools.partial(
    pl.kernel,
    out_type=jax.ShapeDtypeStruct((NC, N, D), jnp.float32),
    mesh=_MESH,
    scratch_types=[
        pltpu.VMEM((CHUNK,), jnp.int32),
        pltpu.VMEM((CHUNK,), jnp.int32),
        pltpu.VMEM((CHUNK, D), jnp.float32),
        pltpu.VMEM((CHUNK, D), jnp.float32),
        pltpu.VMEM_SHARED((N, D), jnp.float32),
        pltpu.SemaphoreType.DMA,
        pltpu.SemaphoreType.DMA,
    ],
)
def _sc_scatter(pre_hbm, r_hbm, zeros_hbm, out_hbm,
                idx0, idx1, buf0, buf1, agg, lsem0, lsem1):
    cid = lax.axis_index("c")
    sid = lax.axis_index("s")
    wid = sid * NC + cid
    rows_per = 624
    pltpu.sync_copy(zeros_hbm.at[pl.ds(sid * rows_per, rows_per)],
                    agg.at[pl.ds(sid * rows_per, rows_per)])

    @pl.when(sid == 0)
    def _():
        pltpu.sync_copy(zeros_hbm.at[pl.ds(NS * rows_per, N - NS * rows_per)],
                        agg.at[pl.ds(NS * rows_per, N - NS * rows_per)])

    plsc.subcore_barrier()

    cnt = SBASE + jnp.where(wid < SREM, 1, 0)
    start = wid * SBASE + jnp.minimum(wid, SREM)
    idx = (idx0, idx1)
    buf = (buf0, buf1)
    lsem = (lsem0, lsem1)

    def fire(i, b):
        row = (start + i) * CHUNK
        pltpu.async_copy(r_hbm.at[pl.ds(row, CHUNK)], idx[b], lsem[b])
        pltpu.async_copy(pre_hbm.at[pl.ds(row, CHUNK)], buf[b], lsem[b])

    def wait_load(b):
        pltpu.make_async_copy(r_hbm.at[pl.ds(0, CHUNK)], idx[b],
                              lsem[b]).wait()
        pltpu.make_async_copy(pre_hbm.at[pl.ds(0, CHUNK)], buf[b],
                              lsem[b]).wait()

    for b in range(2):
        fire(jnp.int32(b), b)

    def group(g, carry):
        for b in range(2):
            i = g * 2 + b

            @pl.when(i < cnt)
            def _():
                wait_load(b)
                pltpu.sync_copy(buf[b], agg.at[idx[b]], add=True)

            nxt = i + 2

            @pl.when(nxt < cnt)
            def _():
                fire(nxt, b)
        return carry

    lax.fori_loop(0, SGROUPS, group, 0)
    plsc.subcore_barrier()

    @pl.when(sid == 0)
    def _():
        pltpu.sync_copy(agg, out_hbm.at[cid])


def _node_body(node_ref, *rest):
    agg_refs = rest[:K]
    w1_ref, b1_ref, w2_ref, b2_ref, g_ref, b_ref, out_ref = rest[K:]
    x = node_ref[...]
    a = agg_refs[0][0] + agg_refs[0][1]
    for r in agg_refs[1:]:
        a = a + r[0] + r[1]
    h1 = (jnp.dot(x, w1_ref[0:D, :], preferred_element_type=jnp.float32)
          + jnp.dot(a, w1_ref[D:2 * D, :], preferred_element_type=jnp.float32)
          + b1_ref[...])
    h1 = jnp.maximum(h1, 0.0)
    h2 = jnp.dot(h1, w2_ref[...], preferred_element_type=jnp.float32) + b2_ref[...]
    mu = jnp.mean(h2, axis=-1, keepdims=True)
    var = jnp.mean((h2 - mu) ** 2, axis=-1, keepdims=True)
    y = (h2 - mu) / jnp.sqrt(var + 1e-5) * g_ref[...] + b_ref[...]
    out_ref[...] = y + x


def _node_call(node, aggs, w1, b1, w2, b2, g, b):
    full = lambda i: (0, 0)
    blk = lambda i: (i, 0)
    agg_spec = pl.BlockSpec((NC, BN, D), lambda i: (0, i, 0))
    return pl.pallas_call(
        _node_body,
        grid=(N // BN,),
        in_specs=[
            pl.BlockSpec((BN, D), blk),
            *([agg_spec] * K),
            pl.BlockSpec((2 * D, H), full),
            pl.BlockSpec((1, H), full),
            pl.BlockSpec((H, D), full),
            pl.BlockSpec((1, D), full),
            pl.BlockSpec((1, D), full),
            pl.BlockSpec((1, D), full),
        ],
        out_specs=pl.BlockSpec((BN, D), blk),
        out_shape=jax.ShapeDtypeStruct((N, D), jnp.float32),
    )(node, *aggs, w1, b1, w2, b2, g, b)


def kernel(node_features, edge_features, W1e, b1e, W2e, b2e, ge, be,
           W1n, b1n, W2n, b2n, gn, bn, senders, receivers):
    ts, tr = _pre_call(node_features, W1e[0:2 * D])
    lo_cols = jnp.concatenate(
        [jnp.arange(32 * c, 32 * c + 16) for c in range(H // 32)])
    perm = jnp.concatenate([lo_cols, lo_cols + 16])
    w1x_p = W1e[2 * D:][:, perm]
    b1e_p = b1e[perm]
    w2e_p = W2e[perm, :]

    zeros = jnp.zeros((N, D), jnp.float32)
    b1e_r = b1e_p.reshape(1, H)
    b2e_r = b2e.reshape(1, D)
    ge_r = ge.reshape(1, D)
    be_r = be.reshape(1, D)

    gsums = [
        _sc_gather(ts, tr, senders[k * EK:(k + 1) * EK],
                   receivers[k * EK:(k + 1) * EK])
        for k in range(K)
    ]
    new_edge = None
    aggs = []
    for k in range(K):
        pre_k, new_edge = _edge_call(
            k, gsums[k], lax.dynamic_slice_in_dim(edge_features, k * EK, EK),
            w1x_p, b1e_r, w2e_p, b2e_r, ge_r, be_r, new_edge)
        aggs.append(_sc_scatter(pre_k, receivers[k * EK:(k + 1) * EK], zeros))

    new_node = _node_call(
        node_features, aggs, W1n, b1n.reshape(1, H), W2n,
        b2n.reshape(1, D), gn.reshape(1, D), bn.reshape(1, D))
    return new_node, new_edge


# speedup vs baseline: 1.0753x; 1.0753x over previous
import functools

import jax
import jax.numpy as jnp
from jax import lax
from jax.experimental import pallas as pl
from jax.experimental.pallas import tpu as pltpu
from jax.experimental.pallas import tpu_sc as plsc

N = 10000
E = 320000
D = 128
H = 128

NC = 2
NS = 16
NW = NC * NS
CHUNK = 128

K = 1
EK = E // K
NCHUNK = EK // CHUNK

GCPT = (((NCHUNK + NW - 1) // NW) + 1) // 2 * 2
GID = GCPT * CHUNK
SBASE = NCHUNK // NW
SREM = NCHUNK % NW
SGROUPS = (SBASE + 2) // 2

BN = 2000
BE = 2000

_MESH = plsc.VectorSubcoreMesh(
    core_axis_name="c", subcore_axis_name="s", num_cores=NC, num_subcores=NS)


def _pre_body(node_ref, w_ref, ps_ref, pr_ref):
    x = node_ref[...]
    ps_ref[...] = jnp.dot(x, w_ref[0:D, :], preferred_element_type=jnp.float32)
    pr_ref[...] = jnp.dot(x, w_ref[D:2 * D, :], preferred_element_type=jnp.float32)


def _pre_call(node, w_sr):
    return pl.pallas_call(
        _pre_body,
        grid=(N // BN,),
        in_specs=[
            pl.BlockSpec((BN, D), lambda i: (i, 0)),
            pl.BlockSpec((2 * D, H), lambda i: (0, 0)),
        ],
        out_specs=[
            pl.BlockSpec((BN, H), lambda i: (i, 0)),
            pl.BlockSpec((BN, H), lambda i: (i, 0)),
        ],
        out_shape=[
            jax.ShapeDtypeStruct((N, H), jnp.float32),
            jax.ShapeDtypeStruct((N, H), jnp.float32),
        ],
    )(node, w_sr)


HW = H // 2


@functools.partial(
    pl.kernel,
    out_type=jax.ShapeDtypeStruct((EK, HW), jnp.int32),
    mesh=_MESH,
    scratch_types=[
        pltpu.VMEM((GID,), jnp.int32),
        pltpu.VMEM((GID,), jnp.int32),
        pltpu.VMEM((CHUNK, H), jnp.float32),
        pltpu.VMEM((CHUNK, H), jnp.float32),
        pltpu.VMEM((CHUNK, HW), jnp.int32),
        pltpu.VMEM((CHUNK, H), jnp.float32),
        pltpu.VMEM((CHUNK, H), jnp.float32),
        pltpu.VMEM((CHUNK, HW), jnp.int32),
        pltpu.SemaphoreType.DMA,
        pltpu.SemaphoreType.DMA,
        pltpu.SemaphoreType.DMA,
        pltpu.SemaphoreType.DMA,
    ],
)
def _sc_gather(ps_hbm, pr_hbm, s_hbm, r_hbm, g_hbm,
               idxs, idxr, bufa0, bufb0, out0, bufa1, bufb1, out1,
               gsem0, gsem1, wsem0, wsem1):
    wid = lax.axis_index("s") * NC + lax.axis_index("c")
    start = (wid * (NCHUNK - GCPT)) // (NW - 1)
    pltpu.sync_copy(s_hbm.at[pl.ds(start * CHUNK, GID)], idxs)
    pltpu.sync_copy(r_hbm.at[pl.ds(start * CHUNK, GID)], idxr)

    bufa = (bufa0, bufa1)
    bufb = (bufb0, bufb1)
    outb = (out0, out1)
    gsem = (gsem0, gsem1)
    wsem = (wsem0, wsem1)

    def fire(i, b):
        pltpu.async_copy(ps_hbm.at[idxs.at[pl.ds(i * CHUNK, CHUNK)]],
                         bufa[b], gsem[b])
        pltpu.async_copy(pr_hbm.at[idxr.at[pl.ds(i * CHUNK, CHUNK)]],
                         bufb[b], gsem[b])

    def wait_gather(i, b):
        pltpu.make_async_copy(ps_hbm.at[idxs.at[pl.ds(i * CHUNK, CHUNK)]],
                              bufa[b], gsem[b]).wait()
        pltpu.make_async_copy(pr_hbm.at[idxr.at[pl.ds(i * CHUNK, CHUNK)]],
                              bufb[b], gsem[b]).wait()

    for b in range(2):
        fire(jnp.int32(b), b)

    def group(g, carry):
        for b in range(2):
            i = g * 2 + b
            wait_gather(i, b)

            @pl.when(g > 0)
            def _():
                pltpu.make_async_copy(
                    outb[b], g_hbm.at[pl.ds(0, CHUNK)], wsem[b]).wait()

            ob, ba, bb = outb[b], bufa[b], bufb[b]

            MHI = jnp.int32(-65536)
            RB = jnp.int32(32768)

            @plsc.parallel_loop(0, CHUNK, 1, unroll=4)
            def _(r):
                for c in range(HW // 16):
                    s0 = pl.ds(32 * c, 16)
                    s1 = pl.ds(32 * c + 16, 16)
                    a0 = lax.bitcast_convert_type(
                        ba[r, s0] + bb[r, s0], jnp.int32) + RB
                    a1 = lax.bitcast_convert_type(
                        ba[r, s1] + bb[r, s1], jnp.int32) + RB
                    ob[r, pl.ds(c * 16, 16)] = jnp.bitwise_or(
                        lax.shift_right_logical(a0, 16),
                        jnp.bitwise_and(a1, MHI))

            nxt = i + 2

            @pl.when(nxt < GCPT)
            def _():
                fire(nxt, b)

            pltpu.async_copy(
                ob, g_hbm.at[pl.ds((start + i) * CHUNK, CHUNK)], wsem[b])
        return carry

    lax.fori_loop(0, GCPT // 2, group, 0)
    for b in range(2):
        pltpu.make_async_copy(outb[b], g_hbm.at[pl.ds(0, CHUNK)],
                              wsem[b]).wait()


def _edge_body(gsum_ref, e_ref, w1x_ref, b1_ref, w2_ref, b2_ref,
               g_ref, b_ref, pre_ref, out_ref):
    e = e_ref[...]
    gi = gsum_ref[...]
    ev = lax.bitcast_convert_type(jnp.left_shift(gi, 16), jnp.float32)
    od = lax.bitcast_convert_type(
        jnp.bitwise_and(gi, jnp.int32(-65536)), jnp.float32)
    gs = jnp.concatenate([ev, od], axis=1)
    h1 = (gs
          + jnp.dot(e, w1x_ref[...], preferred_element_type=jnp.float32)
          + b1_ref[...])
    h1 = jnp.maximum(h1, 0.0)
    h2 = jnp.dot(h1, w2_ref[...], preferred_element_type=jnp.float32) + b2_ref[...]
    mu = jnp.mean(h2, axis=-1, keepdims=True)
    var = jnp.mean((h2 - mu) ** 2, axis=-1, keepdims=True)
    y = (h2 - mu) / jnp.sqrt(var + 1e-5) * g_ref[...] + b_ref[...]
    pre_ref[...] = y
    out_ref[...] = y + e


def _edge_body_alias(gsum_ref, e_ref, w1x_ref, b1_ref, w2_ref, b2_ref,
                     g_ref, b_ref, prev_ref, pre_ref, out_ref):
    del prev_ref
    _edge_body(gsum_ref, e_ref, w1x_ref, b1_ref, w2_ref, b2_ref,
               g_ref, b_ref, pre_ref, out_ref)


def _edge_call(k, gsum, edge_k, w1x, b1, w2, b2, g, b, prev):
    full = lambda i: (0, 0)
    blk = lambda i: (i, 0)
    off = k * (EK // BE)
    out_blk = lambda i, _off=off: (_off + i, 0)
    in_specs = [
        pl.BlockSpec((BE, HW), blk),
        pl.BlockSpec((BE, D), blk),
        pl.BlockSpec((D, H), full),
        pl.BlockSpec((1, H), full),
        pl.BlockSpec((H, D), full),
        pl.BlockSpec((1, D), full),
        pl.BlockSpec((1, D), full),
        pl.BlockSpec((1, D), full),
    ]
    args = [gsum, edge_k, w1x, b1, w2, b2, g, b]
    body = _edge_body
    io_aliases = {}
    if prev is not None:
        in_specs.append(pl.BlockSpec((8, D), full))
        args.append(prev)
        body = _edge_body_alias
        io_aliases = {8: 1}
    return pl.pallas_call(
        body,
        grid=(EK // BE,),
        in_specs=in_specs,
        out_specs=[
            pl.BlockSpec((BE, D), blk),
            pl.BlockSpec((BE, D), out_blk),
        ],
        out_shape=[
            jax.ShapeDtypeStruct((EK, D), jnp.float32),
            jax.ShapeDtypeStruct((E, D), jnp.float32),
        ],
        input_output_aliases=io_aliases,
    )(*args)


@functools.partial(
    pl.kernel,
    out_type=jax.ShapeDtypeStruct((NC, N, D), jnp.float32),
    mesh=_MESH,
    scratch_types=[
        pltpu.VMEM((CHUNK,), jnp.int32),
        pltpu.VMEM((CHUNK,), jnp.int32),
        pltpu.VMEM((CHUNK, D), jnp.float32),
        pltpu.VMEM((CHUNK, D), jnp.float32),
        pltpu.VMEM_SHARED((N, D), jnp.float32),
        pltpu.SemaphoreType.DMA,
        pltpu.SemaphoreType.DMA,
    ],
)
def _sc_scatter(pre_hbm, r_hbm, zeros_hbm, out_hbm,
                idx0, idx1, buf0, buf1, agg, lsem0, lsem1):
    cid = lax.axis_index("c")
    sid = lax.axis_index("s")
    wid = sid * NC + cid
    rows_per = 624
    pltpu.sync_copy(zeros_hbm.at[pl.ds(sid * rows_per, rows_per)],
                    agg.at[pl.ds(sid * rows_per, rows_per)])

    @pl.when(sid == 0)
    def _():
        pltpu.sync_copy(zeros_hbm.at[pl.ds(NS * rows_per, N - NS * rows_per)],
                        agg.at[pl.ds(NS * rows_per, N - NS * rows_per)])

    plsc.subcore_barrier()

    cnt = SBASE + jnp.where(wid < SREM, 1, 0)
    start = wid * SBASE + jnp.minimum(wid, SREM)
    idx = (idx0, idx1)
    buf = (buf0, buf1)
    lsem = (lsem0, lsem1)

    def fire(i, b):
        row = (start + i) * CHUNK
        pltpu.async_copy(r_hbm.at[pl.ds(row, CHUNK)], idx[b], lsem[b])
        pltpu.async_copy(pre_hbm.at[pl.ds(row, CHUNK)], buf[b], lsem[b])

    def wait_load(b):
        pltpu.make_async_copy(r_hbm.at[pl.ds(0, CHUNK)], idx[b],
                              lsem[b]).wait()
        pltpu.make_async_copy(pre_hbm.at[pl.ds(0, CHUNK)], buf[b],
                              lsem[b]).wait()

    for b in range(2):
        fire(jnp.int32(b), b)

    def group(g, carry):
        for b in range(2):
            i = g * 2 + b

            @pl.when(i < cnt)
            def _():
                wait_load(b)
                pltpu.sync_copy(buf[b], agg.at[idx[b]], add=True)

            nxt = i + 2

            @pl.when(nxt < cnt)
            def _():
                fire(nxt, b)
        return carry

    lax.fori_loop(0, SGROUPS, group, 0)
    plsc.subcore_barrier()

    @pl.when(sid == 0)
    def _():
        pltpu.sync_copy(agg, out_hbm.at[cid])


def _node_body(node_ref, *rest):
    agg_refs = rest[:K]
    w1_ref, b1_ref, w2_ref, b2_ref, g_ref, b_ref, out_ref = rest[K:]
    x = node_ref[...]
    a = agg_refs[0][0] + agg_refs[0][1]
    for r in agg_refs[1:]:
        a = a + r[0] + r[1]
    h1 = (jnp.dot(x, w1_ref[0:D, :], preferred_element_type=jnp.float32)
          + jnp.dot(a, w1_ref[D:2 * D, :], preferred_element_type=jnp.float32)
          + b1_ref[...])
    h1 = jnp.maximum(h1, 0.0)
    h2 = jnp.dot(h1, w2_ref[...], preferred_element_type=jnp.float32) + b2_ref[...]
    mu = jnp.mean(h2, axis=-1, keepdims=True)
    var = jnp.mean((h2 - mu) ** 2, axis=-1, keepdims=True)
    y = (h2 - mu) / jnp.sqrt(var + 1e-5) * g_ref[...] + b_ref[...]
    out_ref[...] = y + x


def _node_call(node, aggs, w1, b1, w2, b2, g, b):
    full = lambda i: (0, 0)
    blk = lambda i: (i, 0)
    agg_spec = pl.BlockSpec((NC, BN, D), lambda i: (0, i, 0))
    return pl.pallas_call(
        _node_body,
        grid=(N // BN,),
        in_specs=[
            pl.BlockSpec((BN, D), blk),
            *([agg_spec] * K),
            pl.BlockSpec((2 * D, H), full),
            pl.BlockSpec((1, H), full),
            pl.BlockSpec((H, D), full),
            pl.BlockSpec((1, D), full),
            pl.BlockSpec((1, D), full),
            pl.BlockSpec((1, D), full),
        ],
        out_specs=pl.BlockSpec((BN, D), blk),
        out_shape=jax.ShapeDtypeStruct((N, D), jnp.float32),
    )(node, *aggs, w1, b1, w2, b2, g, b)


def kernel(node_features, edge_features, W1e, b1e, W2e, b2e, ge, be,
           W1n, b1n, W2n, b2n, gn, bn, senders, receivers):
    ts, tr = _pre_call(node_features, W1e[0:2 * D])
    lo_cols = jnp.concatenate(
        [jnp.arange(32 * c, 32 * c + 16) for c in range(H // 32)])
    perm = jnp.concatenate([lo_cols, lo_cols + 16])
    w1x_p = W1e[2 * D:][:, perm]
    b1e_p = b1e[perm]
    w2e_p = W2e[perm, :]

    zeros = jnp.zeros((N, D), jnp.float32)
    b1e_r = b1e_p.reshape(1, H)
    b2e_r = b2e.reshape(1, D)
    ge_r = ge.reshape(1, D)
    be_r = be.reshape(1, D)

    gsums = [
        _sc_gather(ts, tr, senders[k * EK:(k + 1) * EK],
                   receivers[k * EK:(k + 1) * EK])
        for k in range(K)
    ]
    new_edge = None
    aggs = []
    for k in range(K):
        pre_k, new_edge = _edge_call(
            k, gsums[k], lax.dynamic_slice_in_dim(edge_features, k * EK, EK),
            w1x_p, b1e_r, w2e_p, b2e_r, ge_r, be_r, new_edge)
        aggs.append(_sc_scatter(pre_k, receivers[k * EK:(k + 1) * EK], zeros))

    new_node = _node_call(
        node_features, aggs, W1n, b1n.reshape(1, H), W2n,
        b2n.reshape(1, D), gn.reshape(1, D), bn.reshape(1, D))
    return new_node, new_edge


# speedup vs baseline: 1.0816x; 1.0059x over previous
import functools

import jax
import jax.numpy as jnp
from jax import lax
from jax.experimental import pallas as pl
from jax.experimental.pallas import tpu as pltpu
from jax.experimental.pallas import tpu_sc as plsc

N = 10000
E = 320000
D = 128
H = 128

NC = 2
NS = 16
NW = NC * NS
CHUNK = 128

K = 1
EK = E // K
NCHUNK = EK // CHUNK

GCPT = (((NCHUNK + NW - 1) // NW) + 1) // 2 * 2
GID = GCPT * CHUNK
SBASE = NCHUNK // NW
SREM = NCHUNK % NW
SGROUPS = (SBASE + 2) // 2

BN = 2000
BE = 2000

_MESH = plsc.VectorSubcoreMesh(
    core_axis_name="c", subcore_axis_name="s", num_cores=NC, num_subcores=NS)


def _pre_body(node_ref, w_ref, ps_ref, pr_ref):
    x = node_ref[...]
    ps_ref[...] = jnp.dot(x, w_ref[0:D, :], preferred_element_type=jnp.float32)
    pr_ref[...] = jnp.dot(x, w_ref[D:2 * D, :], preferred_element_type=jnp.float32)


def _pre_call(node, w_sr):
    return pl.pallas_call(
        _pre_body,
        grid=(N // BN,),
        in_specs=[
            pl.BlockSpec((BN, D), lambda i: (i, 0)),
            pl.BlockSpec((2 * D, H), lambda i: (0, 0)),
        ],
        out_specs=[
            pl.BlockSpec((BN, H), lambda i: (i, 0)),
            pl.BlockSpec((BN, H), lambda i: (i, 0)),
        ],
        out_shape=[
            jax.ShapeDtypeStruct((N, H), jnp.float32),
            jax.ShapeDtypeStruct((N, H), jnp.float32),
        ],
    )(node, w_sr)


HW = H // 2


@functools.partial(
    pl.kernel,
    out_type=jax.ShapeDtypeStruct((EK, H), jnp.float32),
    mesh=_MESH,
    scratch_types=[
        pltpu.VMEM((GID,), jnp.int32),
        pltpu.VMEM((GID,), jnp.int32),
        pltpu.VMEM((CHUNK, H), jnp.float32),
        pltpu.VMEM((CHUNK, H), jnp.float32),
        pltpu.VMEM((CHUNK, H), jnp.float32),
        pltpu.VMEM((CHUNK, H), jnp.float32),
        pltpu.VMEM((CHUNK, H), jnp.float32),
        pltpu.VMEM((CHUNK, H), jnp.float32),
        pltpu.SemaphoreType.DMA,
        pltpu.SemaphoreType.DMA,
        pltpu.SemaphoreType.DMA,
        pltpu.SemaphoreType.DMA,
    ],
)
def _sc_gather(ps_hbm, pr_hbm, s_hbm, r_hbm, g_hbm,
               idxs, idxr, bufa0, bufb0, out0, bufa1, bufb1, out1,
               gsem0, gsem1, wsem0, wsem1):
    wid = lax.axis_index("s") * NC + lax.axis_index("c")
    start = (wid * (NCHUNK - GCPT)) // (NW - 1)
    pltpu.sync_copy(s_hbm.at[pl.ds(start * CHUNK, GID)], idxs)
    pltpu.sync_copy(r_hbm.at[pl.ds(start * CHUNK, GID)], idxr)

    bufa = (bufa0, bufa1)
    bufb = (bufb0, bufb1)
    outb = (out0, out1)
    gsem = (gsem0, gsem1)
    wsem = (wsem0, wsem1)

    def fire(i, b):
        pltpu.async_copy(ps_hbm.at[idxs.at[pl.ds(i * CHUNK, CHUNK)]],
                         bufa[b], gsem[b])
        pltpu.async_copy(pr_hbm.at[idxr.at[pl.ds(i * CHUNK, CHUNK)]],
                         bufb[b], gsem[b])

    def wait_gather(i, b):
        pltpu.make_async_copy(ps_hbm.at[idxs.at[pl.ds(i * CHUNK, CHUNK)]],
                              bufa[b], gsem[b]).wait()
        pltpu.make_async_copy(pr_hbm.at[idxr.at[pl.ds(i * CHUNK, CHUNK)]],
                              bufb[b], gsem[b]).wait()

    for b in range(2):
        fire(jnp.int32(b), b)

    def group(g, carry):
        for b in range(2):
            i = g * 2 + b
            wait_gather(i, b)

            @pl.when(g > 0)
            def _():
                pltpu.make_async_copy(
                    outb[b], g_hbm.at[pl.ds(0, CHUNK)], wsem[b]).wait()

            ob, ba, bb = outb[b], bufa[b], bufb[b]

            @plsc.parallel_loop(0, CHUNK, 1, unroll=4)
            def _(r):
                for c in range(H // 16):
                    sl = pl.ds(c * 16, 16)
                    ob[r, sl] = ba[r, sl] + bb[r, sl]

            nxt = i + 2

            @pl.when(nxt < GCPT)
            def _():
                fire(nxt, b)

            pltpu.async_copy(
                ob, g_hbm.at[pl.ds((start + i) * CHUNK, CHUNK)], wsem[b])
        return carry

    lax.fori_loop(0, GCPT // 2, group, 0)
    for b in range(2):
        pltpu.make_async_copy(outb[b], g_hbm.at[pl.ds(0, CHUNK)],
                              wsem[b]).wait()


def _edge_body(gsum_ref, e_ref, w1x_ref, b1_ref, w2_ref, b2_ref,
               g_ref, b_ref, pre_ref, out_ref):
    e = e_ref[...]
    h1 = (gsum_ref[...]
          + jnp.dot(e, w1x_ref[...], preferred_element_type=jnp.float32)
          + b1_ref[...])
    h1 = jnp.maximum(h1, 0.0)
    h2 = jnp.dot(h1, w2_ref[...], preferred_element_type=jnp.float32) + b2_ref[...]
    mu = jnp.mean(h2, axis=-1, keepdims=True)
    var = jnp.mean((h2 - mu) ** 2, axis=-1, keepdims=True)
    y = (h2 - mu) / jnp.sqrt(var + 1e-5) * g_ref[...] + b_ref[...]
    pre_ref[...] = y
    out_ref[...] = y + e


def _edge_body_alias(gsum_ref, e_ref, w1x_ref, b1_ref, w2_ref, b2_ref,
                     g_ref, b_ref, prev_ref, pre_ref, out_ref):
    del prev_ref
    _edge_body(gsum_ref, e_ref, w1x_ref, b1_ref, w2_ref, b2_ref,
               g_ref, b_ref, pre_ref, out_ref)


def _edge_call(k, gsum, edge_k, w1x, b1, w2, b2, g, b, prev):
    full = lambda i: (0, 0)
    blk = lambda i: (i, 0)
    off = k * (EK // BE)
    out_blk = lambda i, _off=off: (_off + i, 0)
    in_specs = [
        pl.BlockSpec((BE, H), blk),
        pl.BlockSpec((BE, D), blk),
        pl.BlockSpec((D, H), full),
        pl.BlockSpec((1, H), full),
        pl.BlockSpec((H, D), full),
        pl.BlockSpec((1, D), full),
        pl.BlockSpec((1, D), full),
        pl.BlockSpec((1, D), full),
    ]
    args = [gsum, edge_k, w1x, b1, w2, b2, g, b]
    body = _edge_body
    io_aliases = {}
    if prev is not None:
        in_specs.append(pl.BlockSpec((8, D), full))
        args.append(prev)
        body = _edge_body_alias
        io_aliases = {8: 1}
    return pl.pallas_call(
        body,
        grid=(EK // BE,),
        in_specs=in_specs,
        out_specs=[
            pl.BlockSpec((BE, D), blk),
            pl.BlockSpec((BE, D), out_blk),
        ],
        out_shape=[
            jax.ShapeDtypeStruct((EK, D), jnp.float32),
            jax.ShapeDtypeStruct((E, D), jnp.float32),
        ],
        input_output_aliases=io_aliases,
    )(*args)


@functools.partial(
    pl.kernel,
    out_type=jax.ShapeDtypeStruct((NC, N, D), jnp.float32),
    mesh=_MESH,
    scratch_types=[
        pltpu.VMEM((CHUNK,), jnp.int32),
        pltpu.VMEM((CHUNK,), jnp.int32),
        pltpu.VMEM((CHUNK, D), jnp.float32),
        pltpu.VMEM((CHUNK, D), jnp.float32),
        pltpu.VMEM_SHARED((N, D), jnp.float32),
        pltpu.SemaphoreType.DMA,
        pltpu.SemaphoreType.DMA,
    ],
)
def _sc_scatter(pre_hbm, r_hbm, zeros_hbm, out_hbm,
                idx0, idx1, buf0, buf1, agg, lsem0, lsem1):
    cid = lax.axis_index("c")
    sid = lax.axis_index("s")
    wid = sid * NC + cid
    rows_per = 624
    pltpu.sync_copy(zeros_hbm.at[pl.ds(sid * rows_per, rows_per)],
                    agg.at[pl.ds(sid * rows_per, rows_per)])

    @pl.when(sid == 0)
    def _():
        pltpu.sync_copy(zeros_hbm.at[pl.ds(NS * rows_per, N - NS * rows_per)],
                        agg.at[pl.ds(NS * rows_per, N - NS * rows_per)])

    plsc.subcore_barrier()

    cnt = SBASE + jnp.where(wid < SREM, 1, 0)
    start = wid * SBASE + jnp.minimum(wid, SREM)
    idx = (idx0, idx1)
    buf = (buf0, buf1)
    lsem = (lsem0, lsem1)

    def fire(i, b):
        row = (start + i) * CHUNK
        pltpu.async_copy(r_hbm.at[pl.ds(row, CHUNK)], idx[b], lsem[b])
        pltpu.async_copy(pre_hbm.at[pl.ds(row, CHUNK)], buf[b], lsem[b])

    def wait_load(b):
        pltpu.make_async_copy(r_hbm.at[pl.ds(0, CHUNK)], idx[b],
                              lsem[b]).wait()
        pltpu.make_async_copy(pre_hbm.at[pl.ds(0, CHUNK)], buf[b],
                              lsem[b]).wait()

    for b in range(2):
        fire(jnp.int32(b), b)

    def group(g, carry):
        for b in range(2):
            i = g * 2 + b

            @pl.when(i < cnt)
            def _():
                wait_load(b)
                pltpu.sync_copy(buf[b], agg.at[idx[b]], add=True)

            nxt = i + 2

            @pl.when(nxt < cnt)
            def _():
                fire(nxt, b)
        return carry

    lax.fori_loop(0, SGROUPS, group, 0)
    plsc.subcore_barrier()

    @pl.when(sid == 0)
    def _():
        pltpu.sync_copy(agg, out_hbm.at[cid])


def _node_body(node_ref, *rest):
    agg_refs = rest[:K]
    w1_ref, b1_ref, w2_ref, b2_ref, g_ref, b_ref, out_ref = rest[K:]
    x = node_ref[...]
    a = agg_refs[0][0] + agg_refs[0][1]
    for r in agg_refs[1:]:
        a = a + r[0] + r[1]
    h1 = (jnp.dot(x, w1_ref[0:D, :], preferred_element_type=jnp.float32)
          + jnp.dot(a, w1_ref[D:2 * D, :], preferred_element_type=jnp.float32)
          + b1_ref[...])
    h1 = jnp.maximum(h1, 0.0)
    h2 = jnp.dot(h1, w2_ref[...], preferred_element_type=jnp.float32) + b2_ref[...]
    mu = jnp.mean(h2, axis=-1, keepdims=True)
    var = jnp.mean((h2 - mu) ** 2, axis=-1, keepdims=True)
    y = (h2 - mu) / jnp.sqrt(var + 1e-5) * g_ref[...] + b_ref[...]
    out_ref[...] = y + x


def _node_call(node, aggs, w1, b1, w2, b2, g, b):
    full = lambda i: (0, 0)
    blk = lambda i: (i, 0)
    agg_spec = pl.BlockSpec((NC, BN, D), lambda i: (0, i, 0))
    return pl.pallas_call(
        _node_body,
        grid=(N // BN,),
        in_specs=[
            pl.BlockSpec((BN, D), blk),
            *([agg_spec] * K),
            pl.BlockSpec((2 * D, H), full),
            pl.BlockSpec((1, H), full),
            pl.BlockSpec((H, D), full),
            pl.BlockSpec((1, D), full),
            pl.BlockSpec((1, D), full),
            pl.BlockSpec((1, D), full),
        ],
        out_specs=pl.BlockSpec((BN, D), blk),
        out_shape=jax.ShapeDtypeStruct((N, D), jnp.float32),
    )(node, *aggs, w1, b1, w2, b2, g, b)


def kernel(node_features, edge_features, W1e, b1e, W2e, b2e, ge, be,
           W1n, b1n, W2n, b2n, gn, bn, senders, receivers):
    ts, tr = _pre_call(node_features, W1e[0:2 * D])
    w1x_p = W1e[2 * D:]
    w2e_p = W2e

    zeros = jnp.zeros((N, D), jnp.float32)
    b1e_r = b1e.reshape(1, H)
    b2e_r = b2e.reshape(1, D)
    ge_r = ge.reshape(1, D)
    be_r = be.reshape(1, D)

    gsums = [
        _sc_gather(ts, tr, senders[k * EK:(k + 1) * EK],
                   receivers[k * EK:(k + 1) * EK])
        for k in range(K)
    ]
    new_edge = None
    aggs = []
    for k in range(K):
        pre_k, new_edge = _edge_call(
            k, gsums[k], lax.dynamic_slice_in_dim(edge_features, k * EK, EK),
            w1x_p, b1e_r, w2e_p, b2e_r, ge_r, be_r, new_edge)
        aggs.append(_sc_scatter(pre_k, receivers[k * EK:(k + 1) * EK], zeros))

    new_node = _node_call(
        node_features, aggs, W1n, b1n.reshape(1, H), W2n,
        b2n.reshape(1, D), gn.reshape(1, D), bn.reshape(1, D))
    return new_node, new_edge


# speedup vs baseline: 1.1677x; 1.0797x over previous
import functools

import jax
import jax.numpy as jnp
from jax import lax
from jax.experimental import pallas as pl
from jax.experimental.pallas import tpu as pltpu
from jax.experimental.pallas import tpu_sc as plsc

N = 10000
E = 320000
D = 128
H = 128

NC = 2
NS = 16
NW = NC * NS
CHUNK = 128

K = 1
EK = E // K
NCHUNK = EK // CHUNK

GCPT = (((NCHUNK + NW - 1) // NW) + 1) // 2 * 2
GID = GCPT * CHUNK
SBASE = NCHUNK // NW
SREM = NCHUNK % NW
SGROUPS = (SBASE + 2) // 2

BN = 2000
BE = 4000

_MESH = plsc.VectorSubcoreMesh(
    core_axis_name="c", subcore_axis_name="s", num_cores=NC, num_subcores=NS)


def _pre_body(node_ref, w_ref, ps_ref, pr_ref):
    x = node_ref[...]
    ps_ref[...] = jnp.dot(x, w_ref[0:D, :], preferred_element_type=jnp.float32)
    pr_ref[...] = jnp.dot(x, w_ref[D:2 * D, :], preferred_element_type=jnp.float32)


def _pre_call(node, w_sr):
    return pl.pallas_call(
        _pre_body,
        grid=(N // BN,),
        in_specs=[
            pl.BlockSpec((BN, D), lambda i: (i, 0)),
            pl.BlockSpec((2 * D, H), lambda i: (0, 0)),
        ],
        out_specs=[
            pl.BlockSpec((BN, H), lambda i: (i, 0)),
            pl.BlockSpec((BN, H), lambda i: (i, 0)),
        ],
        out_shape=[
            jax.ShapeDtypeStruct((N, H), jnp.float32),
            jax.ShapeDtypeStruct((N, H), jnp.float32),
        ],
    )(node, w_sr)


HW = H // 2


@functools.partial(
    pl.kernel,
    out_type=jax.ShapeDtypeStruct((EK, H), jnp.float32),
    mesh=_MESH,
    scratch_types=[
        pltpu.VMEM((GID,), jnp.int32),
        pltpu.VMEM((GID,), jnp.int32),
        pltpu.VMEM((CHUNK, H), jnp.float32),
        pltpu.VMEM((CHUNK, H), jnp.float32),
        pltpu.VMEM((CHUNK, H), jnp.float32),
        pltpu.VMEM((CHUNK, H), jnp.float32),
        pltpu.VMEM((CHUNK, H), jnp.float32),
        pltpu.VMEM((CHUNK, H), jnp.float32),
        pltpu.SemaphoreType.DMA,
        pltpu.SemaphoreType.DMA,
        pltpu.SemaphoreType.DMA,
        pltpu.SemaphoreType.DMA,
    ],
)
def _sc_gather(ps_hbm, pr_hbm, s_hbm, r_hbm, g_hbm,
               idxs, idxr, bufa0, bufb0, out0, bufa1, bufb1, out1,
               gsem0, gsem1, wsem0, wsem1):
    wid = lax.axis_index("s") * NC + lax.axis_index("c")
    start = (wid * (NCHUNK - GCPT)) // (NW - 1)
    pltpu.sync_copy(s_hbm.at[pl.ds(start * CHUNK, GID)], idxs)
    pltpu.sync_copy(r_hbm.at[pl.ds(start * CHUNK, GID)], idxr)

    bufa = (bufa0, bufa1)
    bufb = (bufb0, bufb1)
    outb = (out0, out1)
    gsem = (gsem0, gsem1)
    wsem = (wsem0, wsem1)

    def fire(i, b):
        pltpu.async_copy(ps_hbm.at[idxs.at[pl.ds(i * CHUNK, CHUNK)]],
                         bufa[b], gsem[b])
        pltpu.async_copy(pr_hbm.at[idxr.at[pl.ds(i * CHUNK, CHUNK)]],
                         bufb[b], gsem[b])

    def wait_gather(i, b):
        pltpu.make_async_copy(ps_hbm.at[idxs.at[pl.ds(i * CHUNK, CHUNK)]],
                              bufa[b], gsem[b]).wait()
        pltpu.make_async_copy(pr_hbm.at[idxr.at[pl.ds(i * CHUNK, CHUNK)]],
                              bufb[b], gsem[b]).wait()

    for b in range(2):
        fire(jnp.int32(b), b)

    def group(g, carry):
        for b in range(2):
            i = g * 2 + b
            wait_gather(i, b)

            @pl.when(g > 0)
            def _():
                pltpu.make_async_copy(
                    outb[b], g_hbm.at[pl.ds(0, CHUNK)], wsem[b]).wait()

            ob, ba, bb = outb[b], bufa[b], bufb[b]

            @plsc.parallel_loop(0, CHUNK, 1, unroll=4)
            def _(r):
                for c in range(H // 16):
                    sl = pl.ds(c * 16, 16)
                    ob[r, sl] = ba[r, sl] + bb[r, sl]

            nxt = i + 2

            @pl.when(nxt < GCPT)
            def _():
                fire(nxt, b)

            pltpu.async_copy(
                ob, g_hbm.at[pl.ds((start + i) * CHUNK, CHUNK)], wsem[b])
        return carry

    lax.fori_loop(0, GCPT // 2, group, 0)
    for b in range(2):
        pltpu.make_async_copy(outb[b], g_hbm.at[pl.ds(0, CHUNK)],
                              wsem[b]).wait()


def _edge_body(gsum_ref, e_ref, w1x_ref, b1_ref, w2_ref, b2_ref,
               g_ref, b_ref, ones_ref, pre_ref, out_ref):
    e = e_ref[...]
    h1 = (gsum_ref[...]
          + jnp.dot(e, w1x_ref[...], preferred_element_type=jnp.float32)
          + b1_ref[...])
    h1 = jnp.maximum(h1, 0.0)
    h2 = jnp.dot(h1, w2_ref[...], preferred_element_type=jnp.float32) + b2_ref[...]
    inv = 1.0 / D
    mu = jnp.dot(h2, ones_ref[...], preferred_element_type=jnp.float32)[:, 0:1] * inv
    m2 = jnp.dot(h2 * h2, ones_ref[...], preferred_element_type=jnp.float32)[:, 0:1] * inv
    var = m2 - mu * mu
    y = (h2 - mu) * (jax.lax.rsqrt(var + 1e-5) * g_ref[...]) + b_ref[...]
    pre_ref[...] = y
    out_ref[...] = y + e


def _edge_body_alias(gsum_ref, e_ref, w1x_ref, b1_ref, w2_ref, b2_ref,
                     g_ref, b_ref, ones_ref, prev_ref, pre_ref, out_ref):
    del prev_ref
    _edge_body(gsum_ref, e_ref, w1x_ref, b1_ref, w2_ref, b2_ref,
               g_ref, b_ref, ones_ref, pre_ref, out_ref)


def _edge_call(k, gsum, edge_k, w1x, b1, w2, b2, g, b, ones, prev):
    full = lambda i: (0, 0)
    blk = lambda i: (i, 0)
    off = k * (EK // BE)
    out_blk = lambda i, _off=off: (_off + i, 0)
    in_specs = [
        pl.BlockSpec((BE, H), blk),
        pl.BlockSpec((BE, D), blk),
        pl.BlockSpec((D, H), full),
        pl.BlockSpec((1, H), full),
        pl.BlockSpec((H, D), full),
        pl.BlockSpec((1, D), full),
        pl.BlockSpec((1, D), full),
        pl.BlockSpec((1, D), full),
        pl.BlockSpec((D, 8), full),
    ]
    args = [gsum, edge_k, w1x, b1, w2, b2, g, b, ones]
    body = _edge_body
    io_aliases = {}
    if prev is not None:
        in_specs.append(pl.BlockSpec((8, D), full))
        args.append(prev)
        body = _edge_body_alias
        io_aliases = {8: 1}
    return pl.pallas_call(
        body,
        grid=(EK // BE,),
        in_specs=in_specs,
        out_specs=[
            pl.BlockSpec((BE, D), blk),
            pl.BlockSpec((BE, D), out_blk),
        ],
        out_shape=[
            jax.ShapeDtypeStruct((EK, D), jnp.float32),
            jax.ShapeDtypeStruct((E, D), jnp.float32),
        ],
        input_output_aliases=io_aliases,
    )(*args)


@functools.partial(
    pl.kernel,
    out_type=jax.ShapeDtypeStruct((NC, N, D), jnp.float32),
    mesh=_MESH,
    scratch_types=[
        pltpu.VMEM((CHUNK,), jnp.int32),
        pltpu.VMEM((CHUNK,), jnp.int32),
        pltpu.VMEM((CHUNK, D), jnp.float32),
        pltpu.VMEM((CHUNK, D), jnp.float32),
        pltpu.VMEM_SHARED((N, D), jnp.float32),
        pltpu.SemaphoreType.DMA,
        pltpu.SemaphoreType.DMA,
    ],
)
def _sc_scatter(pre_hbm, r_hbm, zeros_hbm, out_hbm,
                idx0, idx1, buf0, buf1, agg, lsem0, lsem1):
    cid = lax.axis_index("c")
    sid = lax.axis_index("s")
    wid = sid * NC + cid
    rows_per = 624
    pltpu.sync_copy(zeros_hbm.at[pl.ds(sid * rows_per, rows_per)],
                    agg.at[pl.ds(sid * rows_per, rows_per)])

    @pl.when(sid == 0)
    def _():
        pltpu.sync_copy(zeros_hbm.at[pl.ds(NS * rows_per, N - NS * rows_per)],
                        agg.at[pl.ds(NS * rows_per, N - NS * rows_per)])

    plsc.subcore_barrier()

    cnt = SBASE + jnp.where(wid < SREM, 1, 0)
    start = wid * SBASE + jnp.minimum(wid, SREM)
    idx = (idx0, idx1)
    buf = (buf0, buf1)
    lsem = (lsem0, lsem1)

    def fire(i, b):
        row = (start + i) * CHUNK
        pltpu.async_copy(r_hbm.at[pl.ds(row, CHUNK)], idx[b], lsem[b])
        pltpu.async_copy(pre_hbm.at[pl.ds(row, CHUNK)], buf[b], lsem[b])

    def wait_load(b):
        pltpu.make_async_copy(r_hbm.at[pl.ds(0, CHUNK)], idx[b],
                              lsem[b]).wait()
        pltpu.make_async_copy(pre_hbm.at[pl.ds(0, CHUNK)], buf[b],
                              lsem[b]).wait()

    for b in range(2):
        fire(jnp.int32(b), b)

    def group(g, carry):
        for b in range(2):
            i = g * 2 + b

            @pl.when(i < cnt)
            def _():
                wait_load(b)
                pltpu.sync_copy(buf[b], agg.at[idx[b]], add=True)

            nxt = i + 2

            @pl.when(nxt < cnt)
            def _():
                fire(nxt, b)
        return carry

    lax.fori_loop(0, SGROUPS, group, 0)
    plsc.subcore_barrier()

    @pl.when(sid == 0)
    def _():
        pltpu.sync_copy(agg, out_hbm.at[cid])


def _node_body(node_ref, *rest):
    agg_refs = rest[:K]
    w1_ref, b1_ref, w2_ref, b2_ref, g_ref, b_ref, out_ref = rest[K:]
    x = node_ref[...]
    a = agg_refs[0][0] + agg_refs[0][1]
    for r in agg_refs[1:]:
        a = a + r[0] + r[1]
    h1 = (jnp.dot(x, w1_ref[0:D, :], preferred_element_type=jnp.float32)
          + jnp.dot(a, w1_ref[D:2 * D, :], preferred_element_type=jnp.float32)
          + b1_ref[...])
    h1 = jnp.maximum(h1, 0.0)
    h2 = jnp.dot(h1, w2_ref[...], preferred_element_type=jnp.float32) + b2_ref[...]
    mu = jnp.mean(h2, axis=-1, keepdims=True)
    var = jnp.mean((h2 - mu) ** 2, axis=-1, keepdims=True)
    y = (h2 - mu) / jnp.sqrt(var + 1e-5) * g_ref[...] + b_ref[...]
    out_ref[...] = y + x


def _node_call(node, aggs, w1, b1, w2, b2, g, b):
    full = lambda i: (0, 0)
    blk = lambda i: (i, 0)
    agg_spec = pl.BlockSpec((NC, BN, D), lambda i: (0, i, 0))
    return pl.pallas_call(
        _node_body,
        grid=(N // BN,),
        in_specs=[
            pl.BlockSpec((BN, D), blk),
            *([agg_spec] * K),
            pl.BlockSpec((2 * D, H), full),
            pl.BlockSpec((1, H), full),
            pl.BlockSpec((H, D), full),
            pl.BlockSpec((1, D), full),
            pl.BlockSpec((1, D), full),
            pl.BlockSpec((1, D), full),
        ],
        out_specs=pl.BlockSpec((BN, D), blk),
        out_shape=jax.ShapeDtypeStruct((N, D), jnp.float32),
    )(node, *aggs, w1, b1, w2, b2, g, b)


def kernel(node_features, edge_features, W1e, b1e, W2e, b2e, ge, be,
           W1n, b1n, W2n, b2n, gn, bn, senders, receivers):
    ts, tr = _pre_call(node_features, W1e[0:2 * D])
    w1x_p = W1e[2 * D:]
    w2e_p = W2e

    zeros = jnp.zeros((N, D), jnp.float32)
    ones = jnp.ones((D, 8), jnp.float32)
    b1e_r = b1e.reshape(1, H)
    b2e_r = b2e.reshape(1, D)
    ge_r = ge.reshape(1, D)
    be_r = be.reshape(1, D)

    gsums = [
        _sc_gather(ts, tr, senders[k * EK:(k + 1) * EK],
                   receivers[k * EK:(k + 1) * EK])
        for k in range(K)
    ]
    new_edge = None
    aggs = []
    for k in range(K):
        pre_k, new_edge = _edge_call(
            k, gsums[k], lax.dynamic_slice_in_dim(edge_features, k * EK, EK),
            w1x_p, b1e_r, w2e_p, b2e_r, ge_r, be_r, ones, new_edge)
        aggs.append(_sc_scatter(pre_k, receivers[k * EK:(k + 1) * EK], zeros))

    new_node = _node_call(
        node_features, aggs, W1n, b1n.reshape(1, H), W2n,
        b2n.reshape(1, D), gn.reshape(1, D), bn.reshape(1, D))
    return new_node, new_edge


# speedup vs baseline: 1.2199x; 1.0447x over previous
import functools

import jax
import jax.numpy as jnp
from jax import lax
from jax.experimental import pallas as pl
from jax.experimental.pallas import tpu as pltpu
from jax.experimental.pallas import tpu_sc as plsc

N = 10000
E = 320000
D = 128
H = 128

NC = 2
NS = 16
NW = NC * NS
CHUNK = 128

K = 1
EK = E // K
NCHUNK = EK // CHUNK

GCPT = (((NCHUNK + NW - 1) // NW) + 1) // 2 * 2
GID = GCPT * CHUNK
SBASE = NCHUNK // NW
SREM = NCHUNK % NW
SGROUPS = (SBASE + 2) // 2

BN = 2000
BE = 8000

_MESH = plsc.VectorSubcoreMesh(
    core_axis_name="c", subcore_axis_name="s", num_cores=NC, num_subcores=NS)


def _pre_body(node_ref, w_ref, ps_ref, pr_ref):
    x = node_ref[...]
    ps_ref[...] = jnp.dot(x, w_ref[0:D, :], preferred_element_type=jnp.float32)
    pr_ref[...] = jnp.dot(x, w_ref[D:2 * D, :], preferred_element_type=jnp.float32)


def _pre_call(node, w_sr):
    return pl.pallas_call(
        _pre_body,
        grid=(N // BN,),
        in_specs=[
            pl.BlockSpec((BN, D), lambda i: (i, 0)),
            pl.BlockSpec((2 * D, H), lambda i: (0, 0)),
        ],
        out_specs=[
            pl.BlockSpec((BN, H), lambda i: (i, 0)),
            pl.BlockSpec((BN, H), lambda i: (i, 0)),
        ],
        out_shape=[
            jax.ShapeDtypeStruct((N, H), jnp.float32),
            jax.ShapeDtypeStruct((N, H), jnp.float32),
        ],
    )(node, w_sr)


HW = H // 2


@functools.partial(
    pl.kernel,
    out_type=jax.ShapeDtypeStruct((EK, H), jnp.float32),
    mesh=_MESH,
    scratch_types=[
        pltpu.VMEM((GID,), jnp.int32),
        pltpu.VMEM((GID,), jnp.int32),
        pltpu.VMEM((CHUNK, H), jnp.float32),
        pltpu.VMEM((CHUNK, H), jnp.float32),
        pltpu.VMEM((CHUNK, H), jnp.float32),
        pltpu.VMEM((CHUNK, H), jnp.float32),
        pltpu.VMEM((CHUNK, H), jnp.float32),
        pltpu.VMEM((CHUNK, H), jnp.float32),
        pltpu.SemaphoreType.DMA,
        pltpu.SemaphoreType.DMA,
        pltpu.SemaphoreType.DMA,
        pltpu.SemaphoreType.DMA,
    ],
)
def _sc_gather(ps_hbm, pr_hbm, s_hbm, r_hbm, g_hbm,
               idxs, idxr, bufa0, bufb0, out0, bufa1, bufb1, out1,
               gsem0, gsem1, wsem0, wsem1):
    wid = lax.axis_index("s") * NC + lax.axis_index("c")
    start = (wid * (NCHUNK - GCPT)) // (NW - 1)
    pltpu.sync_copy(s_hbm.at[pl.ds(start * CHUNK, GID)], idxs)
    pltpu.sync_copy(r_hbm.at[pl.ds(start * CHUNK, GID)], idxr)

    bufa = (bufa0, bufa1)
    bufb = (bufb0, bufb1)
    outb = (out0, out1)
    gsem = (gsem0, gsem1)
    wsem = (wsem0, wsem1)

    def fire(i, b):
        pltpu.async_copy(ps_hbm.at[idxs.at[pl.ds(i * CHUNK, CHUNK)]],
                         bufa[b], gsem[b])
        pltpu.async_copy(pr_hbm.at[idxr.at[pl.ds(i * CHUNK, CHUNK)]],
                         bufb[b], gsem[b])

    def wait_gather(i, b):
        pltpu.make_async_copy(ps_hbm.at[idxs.at[pl.ds(i * CHUNK, CHUNK)]],
                              bufa[b], gsem[b]).wait()
        pltpu.make_async_copy(pr_hbm.at[idxr.at[pl.ds(i * CHUNK, CHUNK)]],
                              bufb[b], gsem[b]).wait()

    for b in range(2):
        fire(jnp.int32(b), b)

    def group(g, carry):
        for b in range(2):
            i = g * 2 + b
            wait_gather(i, b)

            @pl.when(g > 0)
            def _():
                pltpu.make_async_copy(
                    outb[b], g_hbm.at[pl.ds(0, CHUNK)], wsem[b]).wait()

            ob, ba, bb = outb[b], bufa[b], bufb[b]

            @plsc.parallel_loop(0, CHUNK, 1, unroll=4)
            def _(r):
                for c in range(H // 16):
                    sl = pl.ds(c * 16, 16)
                    ob[r, sl] = ba[r, sl] + bb[r, sl]

            nxt = i + 2

            @pl.when(nxt < GCPT)
            def _():
                fire(nxt, b)

            pltpu.async_copy(
                ob, g_hbm.at[pl.ds((start + i) * CHUNK, CHUNK)], wsem[b])
        return carry

    lax.fori_loop(0, GCPT // 2, group, 0)
    for b in range(2):
        pltpu.make_async_copy(outb[b], g_hbm.at[pl.ds(0, CHUNK)],
                              wsem[b]).wait()


def _edge_body(gsum_ref, e_ref, w1x_ref, b1_ref, w2_ref, b2_ref,
               g_ref, b_ref, ones_ref, pre_ref, out_ref):
    e = e_ref[...]
    h1 = (gsum_ref[...]
          + jnp.dot(e, w1x_ref[...], preferred_element_type=jnp.float32)
          + b1_ref[...])
    h1 = jnp.maximum(h1, 0.0)
    h2 = jnp.dot(h1, w2_ref[...], preferred_element_type=jnp.float32) + b2_ref[...]
    inv = 1.0 / D
    mu = jnp.dot(h2, ones_ref[...], preferred_element_type=jnp.float32)[:, 0:1] * inv
    m2 = jnp.dot(h2 * h2, ones_ref[...], preferred_element_type=jnp.float32)[:, 0:1] * inv
    var = m2 - mu * mu
    y = (h2 - mu) * (jax.lax.rsqrt(var + 1e-5) * g_ref[...]) + b_ref[...]
    pre_ref[...] = y
    out_ref[...] = y + e


def _edge_body_alias(gsum_ref, e_ref, w1x_ref, b1_ref, w2_ref, b2_ref,
                     g_ref, b_ref, ones_ref, prev_ref, pre_ref, out_ref):
    del prev_ref
    _edge_body(gsum_ref, e_ref, w1x_ref, b1_ref, w2_ref, b2_ref,
               g_ref, b_ref, ones_ref, pre_ref, out_ref)


def _edge_call(k, gsum, edge_k, w1x, b1, w2, b2, g, b, ones, prev):
    full = lambda i: (0, 0)
    blk = lambda i: (i, 0)
    off = k * (EK // BE)
    out_blk = lambda i, _off=off: (_off + i, 0)
    in_specs = [
        pl.BlockSpec((BE, H), blk),
        pl.BlockSpec((BE, D), blk),
        pl.BlockSpec((D, H), full),
        pl.BlockSpec((1, H), full),
        pl.BlockSpec((H, D), full),
        pl.BlockSpec((1, D), full),
        pl.BlockSpec((1, D), full),
        pl.BlockSpec((1, D), full),
        pl.BlockSpec((D, 8), full),
    ]
    args = [gsum, edge_k, w1x, b1, w2, b2, g, b, ones]
    body = _edge_body
    io_aliases = {}
    if prev is not None:
        in_specs.append(pl.BlockSpec((8, D), full))
        args.append(prev)
        body = _edge_body_alias
        io_aliases = {8: 1}
    return pl.pallas_call(
        body,
        grid=(EK // BE,),
        in_specs=in_specs,
        out_specs=[
            pl.BlockSpec((BE, D), blk),
            pl.BlockSpec((BE, D), out_blk),
        ],
        out_shape=[
            jax.ShapeDtypeStruct((EK, D), jnp.float32),
            jax.ShapeDtypeStruct((E, D), jnp.float32),
        ],
        input_output_aliases=io_aliases,
    )(*args)


@functools.partial(
    pl.kernel,
    out_type=jax.ShapeDtypeStruct((NC, N, D), jnp.float32),
    mesh=_MESH,
    scratch_types=[
        pltpu.VMEM((CHUNK,), jnp.int32),
        pltpu.VMEM((CHUNK,), jnp.int32),
        pltpu.VMEM((CHUNK, D), jnp.float32),
        pltpu.VMEM((CHUNK, D), jnp.float32),
        pltpu.VMEM_SHARED((N, D), jnp.float32),
        pltpu.SemaphoreType.DMA,
        pltpu.SemaphoreType.DMA,
    ],
)
def _sc_scatter(pre_hbm, r_hbm, zeros_hbm, out_hbm,
                idx0, idx1, buf0, buf1, agg, lsem0, lsem1):
    cid = lax.axis_index("c")
    sid = lax.axis_index("s")
    wid = sid * NC + cid
    rows_per = 624
    pltpu.sync_copy(zeros_hbm.at[pl.ds(sid * rows_per, rows_per)],
                    agg.at[pl.ds(sid * rows_per, rows_per)])

    @pl.when(sid == 0)
    def _():
        pltpu.sync_copy(zeros_hbm.at[pl.ds(NS * rows_per, N - NS * rows_per)],
                        agg.at[pl.ds(NS * rows_per, N - NS * rows_per)])

    plsc.subcore_barrier()

    cnt = SBASE + jnp.where(wid < SREM, 1, 0)
    start = wid * SBASE + jnp.minimum(wid, SREM)
    idx = (idx0, idx1)
    buf = (buf0, buf1)
    lsem = (lsem0, lsem1)

    def fire(i, b):
        row = (start + i) * CHUNK
        pltpu.async_copy(r_hbm.at[pl.ds(row, CHUNK)], idx[b], lsem[b])
        pltpu.async_copy(pre_hbm.at[pl.ds(row, CHUNK)], buf[b], lsem[b])

    def wait_load(b):
        pltpu.make_async_copy(r_hbm.at[pl.ds(0, CHUNK)], idx[b],
                              lsem[b]).wait()
        pltpu.make_async_copy(pre_hbm.at[pl.ds(0, CHUNK)], buf[b],
                              lsem[b]).wait()

    for b in range(2):
        fire(jnp.int32(b), b)

    def group(g, carry):
        for b in range(2):
            i = g * 2 + b

            @pl.when(i < cnt)
            def _():
                wait_load(b)
                pltpu.sync_copy(buf[b], agg.at[idx[b]], add=True)

            nxt = i + 2

            @pl.when(nxt < cnt)
            def _():
                fire(nxt, b)
        return carry

    lax.fori_loop(0, SGROUPS, group, 0)
    plsc.subcore_barrier()

    @pl.when(sid == 0)
    def _():
        pltpu.sync_copy(agg, out_hbm.at[cid])


def _node_body(node_ref, *rest):
    agg_refs = rest[:K]
    (w1_ref, b1_ref, w2_ref, b2_ref, g_ref, b_ref, ones_ref,
     out_ref) = rest[K:]
    x = node_ref[...]
    a = agg_refs[0][0] + agg_refs[0][1]
    for r in agg_refs[1:]:
        a = a + r[0] + r[1]
    h1 = (jnp.dot(x, w1_ref[0:D, :], preferred_element_type=jnp.float32)
          + jnp.dot(a, w1_ref[D:2 * D, :], preferred_element_type=jnp.float32)
          + b1_ref[...])
    h1 = jnp.maximum(h1, 0.0)
    h2 = jnp.dot(h1, w2_ref[...], preferred_element_type=jnp.float32) + b2_ref[...]
    inv = 1.0 / D
    mu = jnp.dot(h2, ones_ref[...], preferred_element_type=jnp.float32)[:, 0:1] * inv
    m2 = jnp.dot(h2 * h2, ones_ref[...], preferred_element_type=jnp.float32)[:, 0:1] * inv
    var = m2 - mu * mu
    y = (h2 - mu) * (jax.lax.rsqrt(var + 1e-5) * g_ref[...]) + b_ref[...]
    out_ref[...] = y + x


def _node_call(node, aggs, w1, b1, w2, b2, g, b, ones):
    full = lambda i: (0, 0)
    blk = lambda i: (i, 0)
    agg_spec = pl.BlockSpec((NC, BN, D), lambda i: (0, i, 0))
    return pl.pallas_call(
        _node_body,
        grid=(N // BN,),
        in_specs=[
            pl.BlockSpec((BN, D), blk),
            *([agg_spec] * K),
            pl.BlockSpec((2 * D, H), full),
            pl.BlockSpec((1, H), full),
            pl.BlockSpec((H, D), full),
            pl.BlockSpec((1, D), full),
            pl.BlockSpec((1, D), full),
            pl.BlockSpec((1, D), full),
            pl.BlockSpec((D, 8), full),
        ],
        out_specs=pl.BlockSpec((BN, D), blk),
        out_shape=jax.ShapeDtypeStruct((N, D), jnp.float32),
    )(node, *aggs, w1, b1, w2, b2, g, b, ones)


def kernel(node_features, edge_features, W1e, b1e, W2e, b2e, ge, be,
           W1n, b1n, W2n, b2n, gn, bn, senders, receivers):
    ts, tr = _pre_call(node_features, W1e[0:2 * D])
    w1x_p = W1e[2 * D:]
    w2e_p = W2e

    zeros = jnp.zeros((N, D), jnp.float32)
    ones = jnp.ones((D, 8), jnp.float32)
    b1e_r = b1e.reshape(1, H)
    b2e_r = b2e.reshape(1, D)
    ge_r = ge.reshape(1, D)
    be_r = be.reshape(1, D)

    gsums = [
        _sc_gather(ts, tr, senders[k * EK:(k + 1) * EK],
                   receivers[k * EK:(k + 1) * EK])
        for k in range(K)
    ]
    new_edge = None
    aggs = []
    for k in range(K):
        pre_k, new_edge = _edge_call(
            k, gsums[k], lax.dynamic_slice_in_dim(edge_features, k * EK, EK),
            w1x_p, b1e_r, w2e_p, b2e_r, ge_r, be_r, ones, new_edge)
        aggs.append(_sc_scatter(pre_k, receivers[k * EK:(k + 1) * EK], zeros))

    new_node = _node_call(
        node_features, aggs, W1n, b1n.reshape(1, H), W2n,
        b2n.reshape(1, D), gn.reshape(1, D), bn.reshape(1, D), ones)
    return new_node, new_edge


# speedup vs baseline: 1.2239x; 1.0032x over previous
import functools

import jax
import jax.numpy as jnp
from jax import lax
from jax.experimental import pallas as pl
from jax.experimental.pallas import tpu as pltpu
from jax.experimental.pallas import tpu_sc as plsc

N = 10000
E = 320000
D = 128
H = 128

NC = 2
NS = 16
NW = NC * NS
CHUNK = 128

K = 1
EK = E // K
NCHUNK = EK // CHUNK

GCPT = (((NCHUNK + NW - 1) // NW) + 1) // 2 * 2
GID = GCPT * CHUNK
SBASE = NCHUNK // NW
SREM = NCHUNK % NW
SGROUPS = (SBASE + 2) // 2

BN = 10000
BE = 10000

_MESH = plsc.VectorSubcoreMesh(
    core_axis_name="c", subcore_axis_name="s", num_cores=NC, num_subcores=NS)


def _pre_body(node_ref, w_ref, ps_ref, pr_ref):
    x = node_ref[...]
    ps_ref[...] = jnp.dot(x, w_ref[0:D, :], preferred_element_type=jnp.float32)
    pr_ref[...] = jnp.dot(x, w_ref[D:2 * D, :], preferred_element_type=jnp.float32)


def _pre_call(node, w_sr):
    return pl.pallas_call(
        _pre_body,
        grid=(N // BN,),
        in_specs=[
            pl.BlockSpec((BN, D), lambda i: (i, 0)),
            pl.BlockSpec((2 * D, H), lambda i: (0, 0)),
        ],
        out_specs=[
            pl.BlockSpec((BN, H), lambda i: (i, 0)),
            pl.BlockSpec((BN, H), lambda i: (i, 0)),
        ],
        out_shape=[
            jax.ShapeDtypeStruct((N, H), jnp.float32),
            jax.ShapeDtypeStruct((N, H), jnp.float32),
        ],
    )(node, w_sr)


HW = H // 2


@functools.partial(
    pl.kernel,
    out_type=jax.ShapeDtypeStruct((EK, H), jnp.float32),
    mesh=_MESH,
    scratch_types=[
        pltpu.VMEM((GID,), jnp.int32),
        pltpu.VMEM((GID,), jnp.int32),
        pltpu.VMEM((CHUNK, H), jnp.float32),
        pltpu.VMEM((CHUNK, H), jnp.float32),
        pltpu.VMEM((CHUNK, H), jnp.float32),
        pltpu.VMEM((CHUNK, H), jnp.float32),
        pltpu.VMEM((CHUNK, H), jnp.float32),
        pltpu.VMEM((CHUNK, H), jnp.float32),
        pltpu.SemaphoreType.DMA,
        pltpu.SemaphoreType.DMA,
        pltpu.SemaphoreType.DMA,
        pltpu.SemaphoreType.DMA,
    ],
)
def _sc_gather(ps_hbm, pr_hbm, s_hbm, r_hbm, g_hbm,
               idxs, idxr, bufa0, bufb0, out0, bufa1, bufb1, out1,
               gsem0, gsem1, wsem0, wsem1):
    wid = lax.axis_index("s") * NC + lax.axis_index("c")
    start = (wid * (NCHUNK - GCPT)) // (NW - 1)
    pltpu.sync_copy(s_hbm.at[pl.ds(start * CHUNK, GID)], idxs)
    pltpu.sync_copy(r_hbm.at[pl.ds(start * CHUNK, GID)], idxr)

    bufa = (bufa0, bufa1)
    bufb = (bufb0, bufb1)
    outb = (out0, out1)
    gsem = (gsem0, gsem1)
    wsem = (wsem0, wsem1)

    def fire(i, b):
        pltpu.async_copy(ps_hbm.at[idxs.at[pl.ds(i * CHUNK, CHUNK)]],
                         bufa[b], gsem[b])
        pltpu.async_copy(pr_hbm.at[idxr.at[pl.ds(i * CHUNK, CHUNK)]],
                         bufb[b], gsem[b])

    def wait_gather(i, b):
        pltpu.make_async_copy(ps_hbm.at[idxs.at[pl.ds(i * CHUNK, CHUNK)]],
                              bufa[b], gsem[b]).wait()
        pltpu.make_async_copy(pr_hbm.at[idxr.at[pl.ds(i * CHUNK, CHUNK)]],
                              bufb[b], gsem[b]).wait()

    for b in range(2):
        fire(jnp.int32(b), b)

    def group(g, carry):
        for b in range(2):
            i = g * 2 + b
            wait_gather(i, b)

            @pl.when(g > 0)
            def _():
                pltpu.make_async_copy(
                    outb[b], g_hbm.at[pl.ds(0, CHUNK)], wsem[b]).wait()

            ob, ba, bb = outb[b], bufa[b], bufb[b]

            @plsc.parallel_loop(0, CHUNK, 1, unroll=4)
            def _(r):
                for c in range(H // 16):
                    sl = pl.ds(c * 16, 16)
                    ob[r, sl] = ba[r, sl] + bb[r, sl]

            nxt = i + 2

            @pl.when(nxt < GCPT)
            def _():
                fire(nxt, b)

            pltpu.async_copy(
                ob, g_hbm.at[pl.ds((start + i) * CHUNK, CHUNK)], wsem[b])
        return carry

    lax.fori_loop(0, GCPT // 2, group, 0)
    for b in range(2):
        pltpu.make_async_copy(outb[b], g_hbm.at[pl.ds(0, CHUNK)],
                              wsem[b]).wait()


def _edge_body(gsum_ref, e_ref, w1x_ref, b1_ref, w2_ref, b2_ref,
               g_ref, b_ref, ones_ref, pre_ref, out_ref):
    e = e_ref[...]
    h1 = (gsum_ref[...]
          + jnp.dot(e, w1x_ref[...], preferred_element_type=jnp.float32)
          + b1_ref[...])
    h1 = jnp.maximum(h1, 0.0)
    h2 = jnp.dot(h1, w2_ref[...], preferred_element_type=jnp.float32) + b2_ref[...]
    inv = 1.0 / D
    mu = jnp.dot(h2, ones_ref[...], preferred_element_type=jnp.float32)[:, 0:1] * inv
    m2 = jnp.dot(h2 * h2, ones_ref[...], preferred_element_type=jnp.float32)[:, 0:1] * inv
    var = m2 - mu * mu
    y = (h2 - mu) * (jax.lax.rsqrt(var + 1e-5) * g_ref[...]) + b_ref[...]
    pre_ref[...] = y
    out_ref[...] = y + e


def _edge_body_alias(gsum_ref, e_ref, w1x_ref, b1_ref, w2_ref, b2_ref,
                     g_ref, b_ref, ones_ref, prev_ref, pre_ref, out_ref):
    del prev_ref
    _edge_body(gsum_ref, e_ref, w1x_ref, b1_ref, w2_ref, b2_ref,
               g_ref, b_ref, ones_ref, pre_ref, out_ref)


def _edge_call(k, gsum, edge_k, w1x, b1, w2, b2, g, b, ones, prev):
    full = lambda i: (0, 0)
    blk = lambda i: (i, 0)
    off = k * (EK // BE)
    out_blk = lambda i, _off=off: (_off + i, 0)
    in_specs = [
        pl.BlockSpec((BE, H), blk),
        pl.BlockSpec((BE, D), blk),
        pl.BlockSpec((D, H), full),
        pl.BlockSpec((1, H), full),
        pl.BlockSpec((H, D), full),
        pl.BlockSpec((1, D), full),
        pl.BlockSpec((1, D), full),
        pl.BlockSpec((1, D), full),
        pl.BlockSpec((D, 8), full),
    ]
    args = [gsum, edge_k, w1x, b1, w2, b2, g, b, ones]
    body = _edge_body
    io_aliases = {}
    if prev is not None:
        in_specs.append(pl.BlockSpec((8, D), full))
        args.append(prev)
        body = _edge_body_alias
        io_aliases = {8: 1}
    return pl.pallas_call(
        body,
        grid=(EK // BE,),
        in_specs=in_specs,
        out_specs=[
            pl.BlockSpec((BE, D), blk),
            pl.BlockSpec((BE, D), out_blk),
        ],
        out_shape=[
            jax.ShapeDtypeStruct((EK, D), jnp.float32),
            jax.ShapeDtypeStruct((E, D), jnp.float32),
        ],
        input_output_aliases=io_aliases,
    )(*args)


@functools.partial(
    pl.kernel,
    out_type=jax.ShapeDtypeStruct((NC, N, D), jnp.float32),
    mesh=_MESH,
    scratch_types=[
        pltpu.VMEM((CHUNK,), jnp.int32),
        pltpu.VMEM((CHUNK,), jnp.int32),
        pltpu.VMEM((CHUNK, D), jnp.float32),
        pltpu.VMEM((CHUNK, D), jnp.float32),
        pltpu.VMEM_SHARED((N, D), jnp.float32),
        pltpu.SemaphoreType.DMA,
        pltpu.SemaphoreType.DMA,
    ],
)
def _sc_scatter(pre_hbm, r_hbm, zeros_hbm, out_hbm,
                idx0, idx1, buf0, buf1, agg, lsem0, lsem1):
    cid = lax.axis_index("c")
    sid = lax.axis_index("s")
    wid = sid * NC + cid
    rows_per = 624
    pltpu.sync_copy(zeros_hbm.at[pl.ds(sid * rows_per, rows_per)],
                    agg.at[pl.ds(sid * rows_per, rows_per)])

    @pl.when(sid == 0)
    def _():
        pltpu.sync_copy(zeros_hbm.at[pl.ds(NS * rows_per, N - NS * rows_per)],
                        agg.at[pl.ds(NS * rows_per, N - NS * rows_per)])

    plsc.subcore_barrier()

    cnt = SBASE + jnp.where(wid < SREM, 1, 0)
    start = wid * SBASE + jnp.minimum(wid, SREM)
    idx = (idx0, idx1)
    buf = (buf0, buf1)
    lsem = (lsem0, lsem1)

    def fire(i, b):
        row = (start + i) * CHUNK
        pltpu.async_copy(r_hbm.at[pl.ds(row, CHUNK)], idx[b], lsem[b])
        pltpu.async_copy(pre_hbm.at[pl.ds(row, CHUNK)], buf[b], lsem[b])

    def wait_load(b):
        pltpu.make_async_copy(r_hbm.at[pl.ds(0, CHUNK)], idx[b],
                              lsem[b]).wait()
        pltpu.make_async_copy(pre_hbm.at[pl.ds(0, CHUNK)], buf[b],
                              lsem[b]).wait()

    for b in range(2):
        fire(jnp.int32(b), b)

    def group(g, carry):
        for b in range(2):
            i = g * 2 + b

            @pl.when(i < cnt)
            def _():
                wait_load(b)
                pltpu.sync_copy(buf[b], agg.at[idx[b]], add=True)

            nxt = i + 2

            @pl.when(nxt < cnt)
            def _():
                fire(nxt, b)
        return carry

    lax.fori_loop(0, SGROUPS, group, 0)
    plsc.subcore_barrier()

    @pl.when(sid == 0)
    def _():
        pltpu.sync_copy(agg, out_hbm.at[cid])


def _node_body(node_ref, *rest):
    agg_refs = rest[:K]
    (w1_ref, b1_ref, w2_ref, b2_ref, g_ref, b_ref, ones_ref,
     out_ref) = rest[K:]
    x = node_ref[...]
    a = agg_refs[0][0] + agg_refs[0][1]
    for r in agg_refs[1:]:
        a = a + r[0] + r[1]
    h1 = (jnp.dot(x, w1_ref[0:D, :], preferred_element_type=jnp.float32)
          + jnp.dot(a, w1_ref[D:2 * D, :], preferred_element_type=jnp.float32)
          + b1_ref[...])
    h1 = jnp.maximum(h1, 0.0)
    h2 = jnp.dot(h1, w2_ref[...], preferred_element_type=jnp.float32) + b2_ref[...]
    inv = 1.0 / D
    mu = jnp.dot(h2, ones_ref[...], preferred_element_type=jnp.float32)[:, 0:1] * inv
    m2 = jnp.dot(h2 * h2, ones_ref[...], preferred_element_type=jnp.float32)[:, 0:1] * inv
    var = m2 - mu * mu
    y = (h2 - mu) * (jax.lax.rsqrt(var + 1e-5) * g_ref[...]) + b_ref[...]
    out_ref[...] = y + x


def _node_call(node, aggs, w1, b1, w2, b2, g, b, ones):
    full = lambda i: (0, 0)
    blk = lambda i: (i, 0)
    agg_spec = pl.BlockSpec((NC, BN, D), lambda i: (0, i, 0))
    return pl.pallas_call(
        _node_body,
        grid=(N // BN,),
        in_specs=[
            pl.BlockSpec((BN, D), blk),
            *([agg_spec] * K),
            pl.BlockSpec((2 * D, H), full),
            pl.BlockSpec((1, H), full),
            pl.BlockSpec((H, D), full),
            pl.BlockSpec((1, D), full),
            pl.BlockSpec((1, D), full),
            pl.BlockSpec((1, D), full),
            pl.BlockSpec((D, 8), full),
        ],
        out_specs=pl.BlockSpec((BN, D), blk),
        out_shape=jax.ShapeDtypeStruct((N, D), jnp.float32),
    )(node, *aggs, w1, b1, w2, b2, g, b, ones)


def kernel(node_features, edge_features, W1e, b1e, W2e, b2e, ge, be,
           W1n, b1n, W2n, b2n, gn, bn, senders, receivers):
    ts, tr = _pre_call(node_features, W1e[0:2 * D])
    w1x_p = W1e[2 * D:]
    w2e_p = W2e

    zeros = jnp.zeros((N, D), jnp.float32)
    ones = jnp.ones((D, 8), jnp.float32)
    b1e_r = b1e.reshape(1, H)
    b2e_r = b2e.reshape(1, D)
    ge_r = ge.reshape(1, D)
    be_r = be.reshape(1, D)

    gsums = [
        _sc_gather(ts, tr, senders[k * EK:(k + 1) * EK],
                   receivers[k * EK:(k + 1) * EK])
        for k in range(K)
    ]
    new_edge = None
    aggs = []
    for k in range(K):
        pre_k, new_edge = _edge_call(
            k, gsums[k], lax.dynamic_slice_in_dim(edge_features, k * EK, EK),
            w1x_p, b1e_r, w2e_p, b2e_r, ge_r, be_r, ones, new_edge)
        aggs.append(_sc_scatter(pre_k, receivers[k * EK:(k + 1) * EK], zeros))

    new_node = _node_call(
        node_features, aggs, W1n, b1n.reshape(1, H), W2n,
        b2n.reshape(1, D), gn.reshape(1, D), bn.reshape(1, D), ones)
    return new_node, new_edge
